# baseline (speedup 1.0000x reference)
import functools

import jax
import jax.numpy as jnp
from jax import lax
from jax.experimental import pallas as pl
from jax.experimental.pallas import tpu as pltpu
from jax.experimental.pallas import tpu_sc as plsc

NREAL = 10000
N_PAD = 10240
KNN = 16
OUTC = 64
CPAD = 128
EMB = 512
FEAT = 256
NCLS = 20
EPS = 1e-5
NEG = 0.2

KNN_ROWS = 640
KTILE = 2048
NKT = N_PAD // KTILE
EDGE_ROWS = 1024
HEAD_ROWS = 1024

NWORKERS = 32
GTILE = 128
NPT = N_PAD // GTILE
TPW = KNN * NPT // NWORKERS


CHUNK = 128
NCH = N_PAD // CHUNK
NCAND = KNN * CHUNK


def _knn_a_body(xblk_ref, xallt_ref, cidx_ref, d_ref):
    xblk = xblk_ref[...]
    sq_blk = jnp.sum(xblk * xblk, axis=1, keepdims=True)
    rows = xblk.shape[0]
    lane = lax.broadcasted_iota(jnp.int32, (rows, CHUNK), 1)

    def dcompute(ct, m):
        start = pl.multiple_of(ct * KTILE, KTILE)
        xall_t = xallt_ref[:, pl.ds(start, KTILE)]
        g = lax.dot_general(xblk, xall_t, (((1,), (0,)), ((), ())),
                            preferred_element_type=jnp.float32)
        sq_all = jnp.sum(xall_t * xall_t, axis=0, keepdims=True)
        d = 2.0 * g - sq_blk - sq_all
        col = lax.broadcasted_iota(jnp.int32, d.shape, 1) + ct * KTILE
        d = jnp.where(col < NREAL, d, -jnp.inf)
        d_ref[:, pl.ds(start, KTILE)] = d
        for cc in range(KTILE // CHUNK):
            mc = jnp.max(d[:, cc * CHUNK:(cc + 1) * CHUNK], axis=1,
                         keepdims=True)
            m = jnp.where(lane == ct * (KTILE // CHUNK) + cc, mc, m)
        return m

    m0 = jnp.full((rows, CHUNK), -jnp.inf, jnp.float32)
    m = lax.fori_loop(0, NKT, dcompute, m0, unroll=False)

    m = jnp.where(lane < NCH, m, -jnp.inf)
    lane16 = lax.broadcasted_iota(jnp.int32, (rows, KNN), 1)
    sel = jnp.zeros((rows, CHUNK), jnp.bool_)
    for _ in range(KNN):
        bv = jnp.max(m, axis=1, keepdims=True)
        bi = jnp.min(jnp.where(m == bv, lane, NCH), axis=1, keepdims=True)
        bi = jnp.minimum(bi, NCH - 1)
        sel = sel | (lane == bi)
        m = jnp.where(lane == bi, -jnp.inf, m)
    acc = jnp.zeros((rows, KNN), jnp.int32)
    for k in range(KNN):
        bi = jnp.min(jnp.where(sel, lane, NCH), axis=1, keepdims=True)
        bi = jnp.minimum(bi, NCH - 1)
        acc = jnp.where(lane16 == k, bi, acc)
        sel = sel & (lane != bi)
    cidx_ref[...] = acc


def _knn_a_call(x, xt):
    c = x.shape[1]
    grid = N_PAD // KNN_ROWS
    return pl.pallas_call(
        _knn_a_body,
        grid=(grid,),
        in_specs=[
            pl.BlockSpec((KNN_ROWS, c), lambda i: (i, 0)),
            pl.BlockSpec((c, N_PAD), lambda i: (0, 0)),
        ],
        out_specs=[
            pl.BlockSpec((KNN_ROWS, KNN), lambda i: (i, 0)),
            pl.BlockSpec((KNN_ROWS, N_PAD), lambda i: (i, 0)),
        ],
        out_shape=[
            jax.ShapeDtypeStruct((N_PAD, KNN), jnp.int32),
            jax.ShapeDtypeStruct((N_PAD, N_PAD), jnp.float32),
        ],
    )(x, xt)


CPTS = 8
CTPW = N_PAD // (CPTS * NWORKERS)


def _cand_body(dflat_hbm, ciflat_hbm, dc_hbm,
               ci_v, didx0, didx1, dv0, dv1, sem0, sem1):
    wid = lax.axis_index("s") * 2 + lax.axis_index("c")
    base0 = wid * (CPTS * CTPW)
    didx = (didx0, didx1)
    dv = (dv0, dv1)
    sem = (sem0, sem1)

    def stage(t, buf):
        base = base0 + t * CPTS
        pltpu.sync_copy(ciflat_hbm.at[pl.ds(base * KNN, CPTS * KNN)], ci_v)
        for p in range(CPTS):
            sl = pl.ds(p * KNN, KNN)
            didx[buf][sl] = ci_v[sl] + (base + p) * NCH
        pltpu.async_copy(dflat_hbm.at[didx[buf]], dv[buf], sem[buf])

    def drain(t, buf):
        base = base0 + t * CPTS
        pltpu.make_async_copy(dflat_hbm.at[didx[buf]], dv[buf],
                              sem[buf]).wait()
        pltpu.sync_copy(dv[buf], dc_hbm.at[pl.ds(base * KNN, CPTS * KNN), :])

    stage(0, 0)

    def pair(u, carry):
        t = u * 2
        stage(t + 1, 1)
        drain(t, 0)

        @pl.when(u < CTPW // 2 - 1)
        def _():
            stage(t + 2, 0)

        drain(t + 1, 1)
        return carry

    lax.fori_loop(0, CTPW // 2, pair, 0)


@functools.partial(
    pl.kernel,
    out_type=jax.ShapeDtypeStruct((N_PAD * KNN, CHUNK), jnp.float32),
    mesh=plsc.VectorSubcoreMesh(core_axis_name="c", subcore_axis_name="s",
                                num_cores=2, num_subcores=16),
    scratch_types=[
        pltpu.VMEM((CPTS * KNN,), jnp.int32),
        pltpu.VMEM((CPTS * KNN,), jnp.int32),
        pltpu.VMEM((CPTS * KNN,), jnp.int32),
        pltpu.VMEM((CPTS * KNN, CHUNK), jnp.float32),
        pltpu.VMEM((CPTS * KNN, CHUNK), jnp.float32),
        pltpu.SemaphoreType.DMA,
        pltpu.SemaphoreType.DMA,
    ],
)
def _cand_gather(dflat_hbm, ciflat_hbm, dc_hbm,
                 ci_v, didx0, didx1, dv0, dv1, sem0, sem1):
    _cand_body(dflat_hbm, ciflat_hbm, dc_hbm,
               ci_v, didx0, didx1, dv0, dv1, sem0, sem1)


BTILE = 2048
NBT = NCAND // BTILE


def _knn_b_body(dc_ref, cidx_ref, idx_ref):
    rows = idx_ref.shape[0]
    lane16 = lax.broadcasted_iota(jnp.int32, (rows, KNN), 1)

    def kstep(k, carry):
        acc, prev = carry

        def scan_tile(ct, best):
            bv, bi = best
            start = pl.multiple_of(ct * BTILE, BTILE)
            d = dc_ref[:, pl.ds(start, BTILE)]
            cl = lax.broadcasted_iota(jnp.int32, d.shape, 1) + ct * BTILE
            d = jnp.where(cl == prev, -jnp.inf, d)
            dc_ref[:, pl.ds(start, BTILE)] = d
            tmax = jnp.max(d, axis=1, keepdims=True)
            tidx = jnp.min(jnp.where(d == tmax, cl, NCAND),
                           axis=1, keepdims=True)
            upd = tmax > bv
            return (jnp.where(upd, tmax, bv), jnp.where(upd, tidx, bi))

        bv0 = jnp.full((rows, 1), -jnp.inf, jnp.float32)
        bi0 = jnp.full((rows, 1), NCAND, jnp.int32)
        bv, bi = lax.fori_loop(0, NBT, scan_tile, (bv0, bi0), unroll=False)
        return (jnp.where(lane16 == k, bi, acc), bi)

    acc0 = jnp.zeros((rows, KNN), jnp.int32)
    prev0 = jnp.full((rows, 1), -1, jnp.int32)
    acc, _ = lax.fori_loop(0, KNN, kstep, (acc0, prev0), unroll=False)

    slot = lax.shift_right_logical(acc, 7)
    low = acc & (CHUNK - 1)
    cidx = cidx_ref[...]
    colbase = jnp.zeros((rows, KNN), jnp.int32)
    for s in range(KNN):
        colbase = jnp.where(slot == s, cidx[:, s:s + 1], colbase)
    idx_ref[...] = colbase * CHUNK + low


def _knn_b_call(dc, cidx):
    grid = N_PAD // KNN_ROWS
    return pl.pallas_call(
        _knn_b_body,
        grid=(grid,),
        in_specs=[
            pl.BlockSpec((KNN_ROWS, NCAND), lambda i: (i, 0)),
            pl.BlockSpec((KNN_ROWS, KNN), lambda i: (i, 0)),
        ],
        out_specs=pl.BlockSpec((KNN_ROWS, KNN), lambda i: (i, 0)),
        out_shape=jax.ShapeDtypeStruct((N_PAD, KNN), jnp.int32),
    )(dc, cidx)


def _knn_call(x, xt):
    cidx, dmat = _knn_a_call(x, xt)
    dflat = dmat.reshape(N_PAD * NCH, CHUNK)
    dc = _cand_gather(dflat, cidx.reshape(-1))
    dc = dc.reshape(N_PAD, NCAND)
    return _knn_b_call(dc, cidx)


def _gather_body(tab_hbm, idxt_hbm, out_hbm,
                 idx0, idx1, rows0, rows1, sem0, sem1):
    wid = lax.axis_index("s") * 2 + lax.axis_index("c")
    r = wid // 2
    half = wid % 2
    idx = (idx0, idx1)
    rows = (rows0, rows1)
    sem = (sem0, sem1)

    def stage(t, buf):
        base = (half * TPW + t) * GTILE
        pltpu.sync_copy(idxt_hbm.at[pl.ds(r * N_PAD + base, GTILE)], idx[buf])
        pltpu.async_copy(tab_hbm.at[idx[buf]], rows[buf], sem[buf])

    def drain(t, buf):
        base = (half * TPW + t) * GTILE
        pltpu.make_async_copy(tab_hbm.at[idx[buf]], rows[buf],
                              sem[buf]).wait()
        pltpu.sync_copy(rows[buf], out_hbm.at[r, pl.ds(base, GTILE), :])

    stage(0, 0)

    def pair(u, carry):
        t = u * 2
        stage(t + 1, 1)
        drain(t, 0)

        @pl.when(u < TPW // 2 - 1)
        def _():
            stage(t + 2, 0)

        drain(t + 1, 1)
        return carry

    lax.fori_loop(0, TPW // 2, pair, 0)


@functools.partial(
    pl.kernel,
    out_type=jax.ShapeDtypeStruct((KNN, N_PAD, CPAD), jnp.float32),
    mesh=plsc.VectorSubcoreMesh(core_axis_name="c", subcore_axis_name="s",
                                num_cores=2, num_subcores=16),
    scratch_types=[
        pltpu.VMEM((GTILE,), jnp.int32),
        pltpu.VMEM((GTILE,), jnp.int32),
        pltpu.VMEM((GTILE, CPAD), jnp.float32),
        pltpu.VMEM((GTILE, CPAD), jnp.float32),
        pltpu.SemaphoreType.DMA,
        pltpu.SemaphoreType.DMA,
    ],
)
def _gather_rows(tab_hbm, idxt_hbm, out_hbm,
                 idx0, idx1, rows0, rows1, sem0, sem1):
    _gather_body(tab_hbm, idxt_hbm, out_hbm,
                 idx0, idx1, rows0, rows1, sem0, sem1)


def _edge_body(xg_ref, x_ref, w_ref, b_ref, mean_ref, gs_ref, beta_ref,
               out_ref):
    r = pl.program_id(1)
    xb = x_ref[...]
    cat = jnp.concatenate([xg_ref[0] - xb, xb], axis=1)
    f = jnp.dot(cat, w_ref[...], preferred_element_type=jnp.float32)
    f = f + b_ref[...]
    f = f - mean_ref[...]
    f = f * gs_ref[...]
    f = f + beta_ref[...]
    f = jnp.where(f >= 0, f, NEG * f)

    @pl.when(r == 0)
    def _():
        out_ref[...] = f

    @pl.when(r > 0)
    def _():
        out_ref[...] = jnp.maximum(out_ref[...], f)


def _edge_call(xg, x, w, b, mean, gs, beta):
    grid = (N_PAD // EDGE_ROWS, KNN)
    vspec = pl.BlockSpec((1, CPAD), lambda i, r: (0, 0))
    return pl.pallas_call(
        _edge_body,
        grid=grid,
        in_specs=[
            pl.BlockSpec((1, EDGE_ROWS, CPAD), lambda i, r: (r, i, 0)),
            pl.BlockSpec((EDGE_ROWS, CPAD), lambda i, r: (i, 0)),
            pl.BlockSpec((2 * CPAD, CPAD), lambda i, r: (0, 0)),
            vspec, vspec, vspec, vspec,
        ],
        out_specs=pl.BlockSpec((EDGE_ROWS, CPAD), lambda i, r: (i, 0)),
        out_shape=jax.ShapeDtypeStruct((N_PAD, CPAD), jnp.float32),
    )(xg, x, w, b, mean, gs, beta)


def _head_body(cat_ref, w5t_ref, b5_ref, mean5_ref, gs5_ref, beta5_ref,
               wf0_ref, wf1_ref, wf2_ref,
               af0_ref, af1_ref, af2_ref,
               semw_ref, semb_ref,
               f0_ref, f1_ref, f2_ref, sem_ref):
    x5 = jnp.dot(cat_ref[...], w5t_ref[...],
                 preferred_element_type=jnp.float32)
    x5 = x5 + b5_ref[...]
    x5 = x5 - mean5_ref[...]
    x5 = x5 * gs5_ref[...]
    x5 = x5 + beta5_ref[...]
    x5 = jnp.where(x5 >= 0, x5, NEG * x5)
    ft2 = None
    for wf_ref, af_ref, f_ref in ((wf0_ref, af0_ref, f0_ref),
                                  (wf1_ref, af1_ref, f1_ref),
                                  (wf2_ref, af2_ref, f2_ref)):
        ft = lax.dot_general(wf_ref[...], x5, (((1,), (1,)), ((), ())),
                             preferred_element_type=jnp.float32)
        af = af_ref[...]
        ft = ft + af[:, 0:1]
        ft = ft - af[:, 1:2]
        ft = ft * af[:, 2:3]
        ft = ft + af[:, 3:4]
        f_ref[...] = ft
        ft2 = ft
    sem_ref[...] = (lax.dot_general(ft2, semw_ref[...],
                                    (((0,), (1,)), ((), ())),
                                    preferred_element_type=jnp.float32)
                    + semb_ref[...])


def _head_call(cat, w5t, b5, mean5, gs5, beta5, wfs, afs, semw, semb):
    grid = N_PAD // HEAD_ROWS
    v5spec = pl.BlockSpec((1, EMB), lambda i: (0, 0))
    wfspec = pl.BlockSpec((FEAT, EMB), lambda i: (0, 0))
    afspec = pl.BlockSpec((FEAT, 4), lambda i: (0, 0))
    fspec = pl.BlockSpec((FEAT, HEAD_ROWS), lambda i: (0, i))
    return pl.pallas_call(
        _head_body,
        grid=(grid,),
        in_specs=[pl.BlockSpec((HEAD_ROWS, 4 * OUTC), lambda i: (i, 0)),
                  pl.BlockSpec((4 * OUTC, EMB), lambda i: (0, 0)),
                  v5spec, v5spec, v5spec, v5spec,
                  wfspec, wfspec, wfspec,
                  afspec, afspec, afspec,
                  pl.BlockSpec((NCLS, FEAT), lambda i: (0, 0)),
                  pl.BlockSpec((1, NCLS), lambda i: (0, 0))],
        out_specs=[fspec, fspec, fspec,
                   pl.BlockSpec((HEAD_ROWS, NCLS), lambda i: (i, 0))],
        out_shape=[
            jax.ShapeDtypeStruct((FEAT, N_PAD), jnp.float32),
            jax.ShapeDtypeStruct((FEAT, N_PAD), jnp.float32),
            jax.ShapeDtypeStruct((FEAT, N_PAD), jnp.float32),
            jax.ShapeDtypeStruct((N_PAD, NCLS), jnp.float32),
        ],
    )(cat, w5t, b5, mean5, gs5, beta5, *wfs, *afs, semw, semb)


def _edge_params(conv, bn, cin):
    w = conv["w"]
    wp = jnp.zeros((2 * CPAD, CPAD), jnp.float32)
    wp = wp.at[:cin, :OUTC].set(w[:, :cin].T)
    wp = wp.at[CPAD:CPAD + cin, :OUTC].set(w[:, cin:].T)
    gs = bn["gamma"] / jnp.sqrt(bn["var"] + EPS)
    pad = CPAD - OUTC
    b = jnp.pad(conv["b"], (0, pad))[None, :]
    mean = jnp.pad(bn["mean"], (0, pad))[None, :]
    gsp = jnp.pad(gs, (0, pad), constant_values=1.0)[None, :]
    beta = jnp.pad(bn["beta"], (0, pad))[None, :]
    return wp, b, mean, gsp, beta


def _edge_block(xk, tab, conv, bn, cin):
    idx = _knn_call(xk, xk.T)
    xg = _gather_rows(tab, idx.T.reshape(-1))
    wp, b, mean, gsp, beta = _edge_params(conv, bn, cin)
    return _edge_call(xg, tab, wp, b, mean, gsp, beta)


def kernel(coords, feats, params):
    p = params
    x0 = jnp.concatenate([coords, feats], axis=1)
    xk = jnp.pad(x0, ((0, N_PAD - NREAL), (0, 4)))
    tab = jnp.pad(x0, ((0, N_PAD - NREAL), (0, CPAD - 4)))

    tabs = []
    cin = 4
    for name in ("ec1", "ec2", "ec3", "ec4"):
        tab = _edge_block(xk, tab, p[name], p[name + "_bn"], cin)
        tabs.append(tab)
        xk = tab[:, :OUTC]
        cin = OUTC

    cat = jnp.concatenate([t[:, :OUTC] for t in tabs], axis=1)

    bn5 = p["conv5_bn"]
    gs5 = (bn5["gamma"] / jnp.sqrt(bn5["var"] + EPS))[None, :]
    wfs, afs = [], []
    for i in range(3):
        bno = p["out_bn"][i]
        gso = bno["gamma"] / jnp.sqrt(bno["var"] + EPS)
        wfs.append(p["feat"][i]["w"])
        afs.append(jnp.stack([p["feat"][i]["b"], bno["mean"],
                              gso, bno["beta"]], axis=1))

    f0, f1, f2, sem = _head_call(
        cat, p["conv5"]["w"].T, p["conv5"]["b"][None, :],
        bn5["mean"][None, :], gs5, bn5["beta"][None, :],
        wfs, afs, p["sem_w"], p["sem_b"][None, :])

    ms0 = f0[:, :NREAL][None]
    ms1 = f1[:, :NREAL][None]
    ms2 = f2[:, :NREAL][None]
    sem_logits = sem[:NREAL][None]
    coords_b = coords[None]
    mask = jnp.zeros((1, NREAL), dtype=bool)
    return (ms0, ms1, ms2, coords_b, coords_b, coords_b,
            mask, mask, mask, sem_logits)


# speedup vs baseline: 1.0370x; 1.0370x over previous
import functools

import jax
import jax.numpy as jnp
from jax import lax
from jax.experimental import pallas as pl
from jax.experimental.pallas import tpu as pltpu
from jax.experimental.pallas import tpu_sc as plsc

NREAL = 10000
N_PAD = 10240
KNN = 16
OUTC = 64
CPAD = 128
EMB = 512
FEAT = 256
NCLS = 20
EPS = 1e-5
NEG = 0.2

KNN_ROWS = 640
KTILE = 2048
NKT = N_PAD // KTILE
EDGE_ROWS = 1024
HEAD_ROWS = 1024

NWORKERS = 32
NBUF = 4
GTILE = 128
NPT = N_PAD // GTILE
TPW = KNN * NPT // NWORKERS


CHUNK = 128
NCH = N_PAD // CHUNK
NCAND = KNN * CHUNK


def _knn_a_body(xblk_ref, xallt_ref, cidx_ref, d_ref):
    xblk = xblk_ref[...]
    sq_blk = jnp.sum(xblk * xblk, axis=1, keepdims=True)
    rows = xblk.shape[0]
    lane = lax.broadcasted_iota(jnp.int32, (rows, CHUNK), 1)

    def dcompute(ct, m):
        start = pl.multiple_of(ct * KTILE, KTILE)
        xall_t = xallt_ref[:, pl.ds(start, KTILE)]
        g = lax.dot_general(xblk, xall_t, (((1,), (0,)), ((), ())),
                            preferred_element_type=jnp.float32)
        sq_all = jnp.sum(xall_t * xall_t, axis=0, keepdims=True)
        d = 2.0 * g - sq_blk - sq_all
        col = lax.broadcasted_iota(jnp.int32, d.shape, 1) + ct * KTILE
        d = jnp.where(col < NREAL, d, -jnp.inf)
        d_ref[:, pl.ds(start, KTILE)] = d
        for cc in range(KTILE // CHUNK):
            mc = jnp.max(d[:, cc * CHUNK:(cc + 1) * CHUNK], axis=1,
                         keepdims=True)
            m = jnp.where(lane == ct * (KTILE // CHUNK) + cc, mc, m)
        return m

    m0 = jnp.full((rows, CHUNK), -jnp.inf, jnp.float32)
    m = lax.fori_loop(0, NKT, dcompute, m0, unroll=False)

    m = jnp.where(lane < NCH, m, -jnp.inf)
    lane16 = lax.broadcasted_iota(jnp.int32, (rows, KNN), 1)
    acc = jnp.zeros((rows, KNN), jnp.int32)
    for k in range(KNN):
        bv = jnp.max(m, axis=1, keepdims=True)
        bi = jnp.min(jnp.where(m == bv, lane, NCH), axis=1, keepdims=True)
        bi = jnp.minimum(bi, NCH - 1)
        acc = jnp.where(lane16 == k, bi, acc)
        m = jnp.where(lane == bi, -jnp.inf, m)
    cidx_ref[...] = acc


def _knn_a_call(x, xt):
    c = x.shape[1]
    grid = N_PAD // KNN_ROWS
    return pl.pallas_call(
        _knn_a_body,
        grid=(grid,),
        in_specs=[
            pl.BlockSpec((KNN_ROWS, c), lambda i: (i, 0)),
            pl.BlockSpec((c, N_PAD), lambda i: (0, 0)),
        ],
        out_specs=[
            pl.BlockSpec((KNN_ROWS, KNN), lambda i: (i, 0)),
            pl.BlockSpec((KNN_ROWS, N_PAD), lambda i: (i, 0)),
        ],
        out_shape=[
            jax.ShapeDtypeStruct((N_PAD, KNN), jnp.int32),
            jax.ShapeDtypeStruct((N_PAD, N_PAD), jnp.float32),
        ],
    )(x, xt)


CPTS = 8
CTPW = N_PAD // (CPTS * NWORKERS)


def _cand_body(dflat_hbm, ciflat_hbm, dc_hbm, ci_v, didx, dv, sem):
    wid = lax.axis_index("s") * 2 + lax.axis_index("c")
    base0 = wid * (CPTS * CTPW)

    def stage(t, buf):
        base = base0 + t * CPTS
        pltpu.sync_copy(ciflat_hbm.at[pl.ds(base * KNN, CPTS * KNN)], ci_v)
        for p in range(CPTS):
            sl = pl.ds(p * KNN, KNN)
            didx[buf][sl] = ci_v[sl] + (base + p) * NCH
        pltpu.async_copy(dflat_hbm.at[didx[buf]], dv[buf], sem[buf])

    def drain(t, buf):
        base = base0 + t * CPTS
        pltpu.make_async_copy(dflat_hbm.at[didx[buf]], dv[buf],
                              sem[buf]).wait()
        pltpu.sync_copy(dv[buf], dc_hbm.at[pl.ds(base * KNN, CPTS * KNN), :])

    for b in range(NBUF - 1):
        stage(b, b)

    def group(g, carry):
        t0 = g * NBUF
        for b in range(NBUF):
            t = t0 + b
            s = t + NBUF - 1

            @pl.when(s < CTPW)
            def _():
                stage(s, (b + NBUF - 1) % NBUF)

            drain(t, b)
        return carry

    lax.fori_loop(0, CTPW // NBUF, group, 0)


@functools.partial(
    pl.kernel,
    out_type=jax.ShapeDtypeStruct((N_PAD * KNN, CHUNK), jnp.float32),
    mesh=plsc.VectorSubcoreMesh(core_axis_name="c", subcore_axis_name="s",
                                num_cores=2, num_subcores=16),
    scratch_types=[
        pltpu.VMEM((CPTS * KNN,), jnp.int32),
        [pltpu.VMEM((CPTS * KNN,), jnp.int32)] * NBUF,
        [pltpu.VMEM((CPTS * KNN, CHUNK), jnp.float32)] * NBUF,
        [pltpu.SemaphoreType.DMA] * NBUF,
    ],
)
def _cand_gather(dflat_hbm, ciflat_hbm, dc_hbm, ci_v, didx, dv, sem):
    _cand_body(dflat_hbm, ciflat_hbm, dc_hbm, ci_v, didx, dv, sem)


BTILE = 2048
NBT = NCAND // BTILE


def _knn_b_body(dc_ref, cidx_ref, idx_ref, cc_ref):
    rows = idx_ref.shape[0]
    lane16 = lax.broadcasted_iota(jnp.int32, (rows, KNN), 1)

    cf = cidx_ref[...].astype(jnp.float32)
    slot = lax.broadcasted_iota(jnp.int32, (KNN, NCAND), 0)
    jj = lax.broadcasted_iota(jnp.int32, (KNN, NCAND), 1)
    e = jnp.where(lax.shift_right_logical(jj, 7) == slot,
                  jnp.float32(CHUNK), jnp.float32(0.0))
    colsf = jnp.dot(cf, e, preferred_element_type=jnp.float32)
    low = lax.broadcasted_iota(jnp.int32, (rows, NCAND), 1) & (CHUNK - 1)
    cc_ref[...] = colsf.astype(jnp.int32) + low

    def kstep(k, carry):
        acc, prev = carry

        def scan_tile(ct, best):
            bv, bi = best
            start = pl.multiple_of(ct * BTILE, BTILE)
            d = dc_ref[:, pl.ds(start, BTILE)]
            col = cc_ref[:, pl.ds(start, BTILE)]
            d = jnp.where(col == prev, -jnp.inf, d)
            dc_ref[:, pl.ds(start, BTILE)] = d
            tmax = jnp.max(d, axis=1, keepdims=True)
            tidx = jnp.min(jnp.where(d == tmax, col, N_PAD),
                           axis=1, keepdims=True)
            upd = tmax > bv
            return (jnp.where(upd, tmax, bv), jnp.where(upd, tidx, bi))

        bv0 = jnp.full((rows, 1), -jnp.inf, jnp.float32)
        bi0 = jnp.full((rows, 1), N_PAD, jnp.int32)
        bv, bi = lax.fori_loop(0, NBT, scan_tile, (bv0, bi0), unroll=False)
        return (jnp.where(lane16 == k, bi, acc), bi)

    acc0 = jnp.zeros((rows, KNN), jnp.int32)
    prev0 = jnp.full((rows, 1), -1, jnp.int32)
    acc, _ = lax.fori_loop(0, KNN, kstep, (acc0, prev0), unroll=False)
    idx_ref[...] = acc


def _knn_b_call(dc, cidx):
    grid = N_PAD // KNN_ROWS
    return pl.pallas_call(
        _knn_b_body,
        grid=(grid,),
        in_specs=[
            pl.BlockSpec((KNN_ROWS, NCAND), lambda i: (i, 0)),
            pl.BlockSpec((KNN_ROWS, KNN), lambda i: (i, 0)),
        ],
        out_specs=pl.BlockSpec((KNN_ROWS, KNN), lambda i: (i, 0)),
        out_shape=jax.ShapeDtypeStruct((N_PAD, KNN), jnp.int32),
        scratch_shapes=[pltpu.VMEM((KNN_ROWS, NCAND), jnp.int32)],
    )(dc, cidx)


def _knn_call(x, xt):
    cidx, dmat = _knn_a_call(x, xt)
    dflat = dmat.reshape(N_PAD * NCH, CHUNK)
    dc = _cand_gather(dflat, cidx.reshape(-1))
    dc = dc.reshape(N_PAD, NCAND)
    return _knn_b_call(dc, cidx)


def _gather_body(tab_hbm, idxt_hbm, out_hbm, idx, rows, sem):
    wid = lax.axis_index("s") * 2 + lax.axis_index("c")
    r = wid // 2
    half = wid % 2

    def stage(t, buf):
        base = (half * TPW + t) * GTILE
        pltpu.sync_copy(idxt_hbm.at[pl.ds(r * N_PAD + base, GTILE)], idx[buf])
        pltpu.async_copy(tab_hbm.at[idx[buf]], rows[buf], sem[buf])

    def drain(t, buf):
        base = (half * TPW + t) * GTILE
        pltpu.make_async_copy(tab_hbm.at[idx[buf]], rows[buf],
                              sem[buf]).wait()
        pltpu.sync_copy(rows[buf], out_hbm.at[r, pl.ds(base, GTILE), :])

    for b in range(NBUF - 1):
        stage(b, b)

    def group(g, carry):
        t0 = g * NBUF
        for b in range(NBUF):
            t = t0 + b
            s = t + NBUF - 1

            @pl.when(s < TPW)
            def _():
                stage(s, (b + NBUF - 1) % NBUF)

            drain(t, b)
        return carry

    lax.fori_loop(0, TPW // NBUF, group, 0)


@functools.partial(
    pl.kernel,
    out_type=jax.ShapeDtypeStruct((KNN, N_PAD, CPAD), jnp.float32),
    mesh=plsc.VectorSubcoreMesh(core_axis_name="c", subcore_axis_name="s",
                                num_cores=2, num_subcores=16),
    scratch_types=[
        [pltpu.VMEM((GTILE,), jnp.int32)] * NBUF,
        [pltpu.VMEM((GTILE, CPAD), jnp.float32)] * NBUF,
        [pltpu.SemaphoreType.DMA] * NBUF,
    ],
)
def _gather_rows(tab_hbm, idxt_hbm, out_hbm, idx, rows, sem):
    _gather_body(tab_hbm, idxt_hbm, out_hbm, idx, rows, sem)


def _edge_body(xg_ref, x_ref, w_ref, b_ref, mean_ref, gs_ref, beta_ref,
               out_ref):
    r = pl.program_id(1)
    xb = x_ref[...]
    cat = jnp.concatenate([xg_ref[0] - xb, xb], axis=1)
    f = jnp.dot(cat, w_ref[...], preferred_element_type=jnp.float32)
    f = f + b_ref[...]
    f = f - mean_ref[...]
    f = f * gs_ref[...]
    f = f + beta_ref[...]
    f = jnp.where(f >= 0, f, NEG * f)

    @pl.when(r == 0)
    def _():
        out_ref[...] = f

    @pl.when(r > 0)
    def _():
        out_ref[...] = jnp.maximum(out_ref[...], f)


def _edge_call(xg, x, w, b, mean, gs, beta):
    grid = (N_PAD // EDGE_ROWS, KNN)
    vspec = pl.BlockSpec((1, CPAD), lambda i, r: (0, 0))
    return pl.pallas_call(
        _edge_body,
        grid=grid,
        in_specs=[
            pl.BlockSpec((1, EDGE_ROWS, CPAD), lambda i, r: (r, i, 0)),
            pl.BlockSpec((EDGE_ROWS, CPAD), lambda i, r: (i, 0)),
            pl.BlockSpec((2 * CPAD, CPAD), lambda i, r: (0, 0)),
            vspec, vspec, vspec, vspec,
        ],
        out_specs=pl.BlockSpec((EDGE_ROWS, CPAD), lambda i, r: (i, 0)),
        out_shape=jax.ShapeDtypeStruct((N_PAD, CPAD), jnp.float32),
    )(xg, x, w, b, mean, gs, beta)


def _head_body(cat_ref, w5t_ref, b5_ref, mean5_ref, gs5_ref, beta5_ref,
               wf0_ref, wf1_ref, wf2_ref,
               af0_ref, af1_ref, af2_ref,
               semw_ref, semb_ref,
               f0_ref, f1_ref, f2_ref, sem_ref):
    x5 = jnp.dot(cat_ref[...], w5t_ref[...],
                 preferred_element_type=jnp.float32)
    x5 = x5 + b5_ref[...]
    x5 = x5 - mean5_ref[...]
    x5 = x5 * gs5_ref[...]
    x5 = x5 + beta5_ref[...]
    x5 = jnp.where(x5 >= 0, x5, NEG * x5)
    ft2 = None
    for wf_ref, af_ref, f_ref in ((wf0_ref, af0_ref, f0_ref),
                                  (wf1_ref, af1_ref, f1_ref),
                                  (wf2_ref, af2_ref, f2_ref)):
        ft = lax.dot_general(wf_ref[...], x5, (((1,), (1,)), ((), ())),
                             preferred_element_type=jnp.float32)
        af = af_ref[...]
        ft = ft + af[:, 0:1]
        ft = ft - af[:, 1:2]
        ft = ft * af[:, 2:3]
        ft = ft + af[:, 3:4]
        f_ref[...] = ft
        ft2 = ft
    sem_ref[...] = (lax.dot_general(ft2, semw_ref[...],
                                    (((0,), (1,)), ((), ())),
                                    preferred_element_type=jnp.float32)
                    + semb_ref[...])


def _head_call(cat, w5t, b5, mean5, gs5, beta5, wfs, afs, semw, semb):
    grid = N_PAD // HEAD_ROWS
    v5spec = pl.BlockSpec((1, EMB), lambda i: (0, 0))
    wfspec = pl.BlockSpec((FEAT, EMB), lambda i: (0, 0))
    afspec = pl.BlockSpec((FEAT, 4), lambda i: (0, 0))
    fspec = pl.BlockSpec((FEAT, HEAD_ROWS), lambda i: (0, i))
    return pl.pallas_call(
        _head_body,
        grid=(grid,),
        in_specs=[pl.BlockSpec((HEAD_ROWS, 4 * OUTC), lambda i: (i, 0)),
                  pl.BlockSpec((4 * OUTC, EMB), lambda i: (0, 0)),
                  v5spec, v5spec, v5spec, v5spec,
                  wfspec, wfspec, wfspec,
                  afspec, afspec, afspec,
                  pl.BlockSpec((NCLS, FEAT), lambda i: (0, 0)),
                  pl.BlockSpec((1, NCLS), lambda i: (0, 0))],
        out_specs=[fspec, fspec, fspec,
                   pl.BlockSpec((HEAD_ROWS, NCLS), lambda i: (i, 0))],
        out_shape=[
            jax.ShapeDtypeStruct((FEAT, N_PAD), jnp.float32),
            jax.ShapeDtypeStruct((FEAT, N_PAD), jnp.float32),
            jax.ShapeDtypeStruct((FEAT, N_PAD), jnp.float32),
            jax.ShapeDtypeStruct((N_PAD, NCLS), jnp.float32),
        ],
    )(cat, w5t, b5, mean5, gs5, beta5, *wfs, *afs, semw, semb)


def _edge_params(conv, bn, cin):
    w = conv["w"]
    wp = jnp.zeros((2 * CPAD, CPAD), jnp.float32)
    wp = wp.at[:cin, :OUTC].set(w[:, :cin].T)
    wp = wp.at[CPAD:CPAD + cin, :OUTC].set(w[:, cin:].T)
    gs = bn["gamma"] / jnp.sqrt(bn["var"] + EPS)
    pad = CPAD - OUTC
    b = jnp.pad(conv["b"], (0, pad))[None, :]
    mean = jnp.pad(bn["mean"], (0, pad))[None, :]
    gsp = jnp.pad(gs, (0, pad), constant_values=1.0)[None, :]
    beta = jnp.pad(bn["beta"], (0, pad))[None, :]
    return wp, b, mean, gsp, beta


def _edge_block(xk, tab, conv, bn, cin):
    idx = _knn_call(xk, xk.T)
    xg = _gather_rows(tab, idx.T.reshape(-1))
    wp, b, mean, gsp, beta = _edge_params(conv, bn, cin)
    return _edge_call(xg, tab, wp, b, mean, gsp, beta)


def kernel(coords, feats, params):
    p = params
    x0 = jnp.concatenate([coords, feats], axis=1)
    xk = jnp.pad(x0, ((0, N_PAD - NREAL), (0, 4)))
    tab = jnp.pad(x0, ((0, N_PAD - NREAL), (0, CPAD - 4)))

    tabs = []
    cin = 4
    for name in ("ec1", "ec2", "ec3", "ec4"):
        tab = _edge_block(xk, tab, p[name], p[name + "_bn"], cin)
        tabs.append(tab)
        xk = tab[:, :OUTC]
        cin = OUTC

    cat = jnp.concatenate([t[:, :OUTC] for t in tabs], axis=1)

    bn5 = p["conv5_bn"]
    gs5 = (bn5["gamma"] / jnp.sqrt(bn5["var"] + EPS))[None, :]
    wfs, afs = [], []
    for i in range(3):
        bno = p["out_bn"][i]
        gso = bno["gamma"] / jnp.sqrt(bno["var"] + EPS)
        wfs.append(p["feat"][i]["w"])
        afs.append(jnp.stack([p["feat"][i]["b"], bno["mean"],
                              gso, bno["beta"]], axis=1))

    f0, f1, f2, sem = _head_call(
        cat, p["conv5"]["w"].T, p["conv5"]["b"][None, :],
        bn5["mean"][None, :], gs5, bn5["beta"][None, :],
        wfs, afs, p["sem_w"], p["sem_b"][None, :])

    ms0 = f0[:, :NREAL][None]
    ms1 = f1[:, :NREAL][None]
    ms2 = f2[:, :NREAL][None]
    sem_logits = sem[:NREAL][None]
    coords_b = coords[None]
    mask = jnp.zeros((1, NREAL), dtype=bool)
    return (ms0, ms1, ms2, coords_b, coords_b, coords_b,
            mask, mask, mask, sem_logits)
